# baseline (speedup 1.0000x reference)
import functools

import jax
import jax.numpy as jnp
from jax import lax
from jax.experimental import pallas as pl
from jax.experimental.pallas import tpu as pltpu
from jax.experimental.pallas import tpu_sc as plsc

BATCH = 16384
EMB = 64
NC = 2
NS = 16
NW = NC * NS
B_PER_W = BATCH // NW
CHUNK = 128
NCHUNK = B_PER_W // CHUNK

TR_W = 4096
TR_SH = TR_W.bit_length() - 1
TR_NBLK = -(-1000000 // TR_W)
NROW2 = TR_NBLK * (TR_W // 2)

_SHUF_DNUMS = lax.GatherDimensionNumbers(
    offset_dims=(), collapsed_slice_dims=(0,), start_index_map=(0,))


def _shuffle(x, perm):
    return lax.gather(x, perm[:, None], _SHUF_DNUMS, slice_sizes=(1,),
                      mode=lax.GatherScatterMode.PROMISE_IN_BOUNDS)


def _tr_body(x_ref, o_ref):
    y = jnp.transpose(x_ref[...])
    o_ref[:, 0:EMB] = y[0:TR_W // 2]
    o_ref[:, EMB:2 * EMB] = y[TR_W // 2:TR_W]


def _transpose_pack(tbl_T):
    return pl.pallas_call(
        _tr_body,
        grid=(TR_NBLK,),
        in_specs=[pl.BlockSpec((EMB, TR_W), lambda j: (0, j))],
        out_specs=pl.BlockSpec((TR_W // 2, 2 * EMB), lambda j: (j, 0)),
        out_shape=jax.ShapeDtypeStruct((NROW2, 2 * EMB), jnp.float32),
    )(tbl_T)


def _body_a(u_hbm, user_hbm, out_hbm,
            u_raw, u_idx, ue, sel, sem0, sem1):
    wid = lax.axis_index("s") * NC + lax.axis_index("c")
    base = wid * B_PER_W

    for j in range(NCHUNK):
        pltpu.sync_copy(u_hbm.at[pl.ds(base + j * CHUNK, CHUNK)], u_raw.at[j])
    for j in range(NCHUNK):
        for t in range(CHUNK // 16):
            sl = pl.ds(t * 16, 16)
            uv = u_raw[j, sl]
            u_idx[j, sl] = (lax.shift_right_logical(uv, TR_SH) * (TR_W // 2) +
                            (uv & (TR_W // 2 - 1)))

    sems = (sem0, sem1)

    def fire(c):
        b = c % 2
        return pltpu.async_copy(user_hbm.at[u_idx.at[c]], ue.at[b], sems[b])

    inflight = fire(0)
    for c in range(NCHUNK):
        nxt = fire(c + 1) if c + 1 < NCHUNK else None
        inflight.wait()
        inflight = nxt
        b = c % 2

        def row_body(g, carry, c=c, b=b):
            r0 = g * 16
            pu_vec = (lax.shift_right_logical(u_raw[c, pl.ds(r0, 16)],
                                              TR_SH - 1) & 1) * EMB
            for k in range(16):
                r = r0 + k
                pu = pu_vec[k]
                for q in range(EMB // 16):
                    sel[c * CHUNK + r, pl.ds(q * 16, 16)] = (
                        ue[b, r, pl.ds(pu + q * 16, 16)])
            return carry

        lax.fori_loop(0, CHUNK // 16, row_body, 0)

    pltpu.sync_copy(sel, out_hbm.at[pl.ds(base, B_PER_W), :])


def _body_b(v_hbm, item_hbm, uesel_hbm, out_hbm,
            v_raw, ue, ve, out_v, sem0, sem1, semu):
    wid = lax.axis_index("s") * NC + lax.axis_index("c")
    base = wid * B_PER_W

    ucopy = pltpu.async_copy(uesel_hbm.at[pl.ds(base, B_PER_W), :], ue, semu)
    for j in range(NCHUNK):
        pltpu.sync_copy(v_hbm.at[pl.ds(base + j * CHUNK, CHUNK)], v_raw.at[j])

    sems = (sem0, sem1)

    def fire(c):
        b = c % 2
        return pltpu.async_copy(item_hbm.at[v_raw.at[c]], ve.at[b], sems[b])

    lanes = lax.iota(jnp.int32, 16)
    zero16 = jnp.zeros((16,), jnp.float32)

    inflight = fire(0)
    ucopy.wait()
    for c in range(NCHUNK):
        nxt = fire(c + 1) if c + 1 < NCHUNK else None
        inflight.wait()
        inflight = nxt
        b = c % 2

        def group_body(g, carry, c=c, b=b):
            r0 = g * 16
            acc = zero16
            for k in range(16):
                r = r0 + k
                ra = c * CHUNK + r
                p = ue[ra, pl.ds(0, 16)] * ve[b, r, pl.ds(0, 16)]
                for q in range(1, EMB // 16):
                    p = p + (ue[ra, pl.ds(q * 16, 16)] *
                             ve[b, r, pl.ds(q * 16, 16)])
                for s in (8, 4, 2, 1):
                    p = p + _shuffle(p, lanes ^ s)
                acc = jnp.where(lanes == k, p, acc)
            out_v[pl.ds(c * CHUNK + r0, 16)] = acc
            return carry

        lax.fori_loop(0, CHUNK // 16, group_body, 0)

    pltpu.sync_copy(out_v, out_hbm.at[pl.ds(base, B_PER_W)])


@jax.jit
def _run(u, v, user_emb, item_emb):
    mesh = plsc.VectorSubcoreMesh(core_axis_name="c", subcore_axis_name="s")
    kern_a = functools.partial(
        pl.kernel,
        mesh=mesh,
        compiler_params=pltpu.CompilerParams(use_tc_tiling_on_sc=True),
        out_type=jax.ShapeDtypeStruct((BATCH, EMB), jnp.float32),
        scratch_types=[
            pltpu.VMEM((NCHUNK, CHUNK), jnp.int32),
            pltpu.VMEM((NCHUNK, CHUNK), jnp.int32),
            pltpu.VMEM((2, CHUNK, 2 * EMB), jnp.float32),
            pltpu.VMEM((B_PER_W, EMB), jnp.float32),
            pltpu.SemaphoreType.DMA,
            pltpu.SemaphoreType.DMA,
        ],
    )(_body_a)
    kern_b = functools.partial(
        pl.kernel,
        mesh=mesh,
        compiler_params=pltpu.CompilerParams(use_tc_tiling_on_sc=False),
        out_type=jax.ShapeDtypeStruct((BATCH,), jnp.float32),
        scratch_types=[
            pltpu.VMEM((NCHUNK, CHUNK), jnp.int32),
            pltpu.VMEM((B_PER_W, EMB), jnp.float32),
            pltpu.VMEM((2, CHUNK, EMB), jnp.float32),
            pltpu.VMEM((B_PER_W,), jnp.float32),
            pltpu.SemaphoreType.DMA,
            pltpu.SemaphoreType.DMA,
            pltpu.SemaphoreType.DMA,
        ],
    )(_body_b)
    user2 = _transpose_pack(user_emb.T)
    ue_sel = kern_a(u, user2)
    return kern_b(v, item_emb, ue_sel)


def kernel(u, v, user_emb, item_emb):
    return _run(u, v, user_emb, item_emb)
